# baseline (speedup 1.0000x reference)
import jax
import jax.numpy as jnp
import numpy as np
from jax import lax
from jax.experimental import pallas as pl
from jax.experimental.pallas import tpu as pltpu
from jax.experimental.pallas import tpu_sc as plsc

NC, NS, L = 2, 16, 16
NW = NC * NS

VOCAB = 100000
BATCH, SEQ = 4096, 200
EMBED, HIDDEN = 128, 128
SEQ_HALF = SEQ // 2
ROWS_PER_W = BATCH // NW
NCHUNK = EMBED // L


def _pool_body(ids_hbm, table_hbm, out_hbm, idx_all, rows_v, acc_v,
               sem0, sem1, osem0, osem1):
    wid = lax.axis_index("s") * NC + lax.axis_index("c")
    base = wid * ROWS_PER_W
    sems = (sem0, sem1)
    osems = (osem0, osem1)

    pltpu.sync_copy(ids_hbm.at[pl.ds(base, ROWS_PER_W)], idx_all)

    def issue(slot, r):
        for j in range(2):
            pltpu.async_copy(table_hbm.at[idx_all.at[r, j]],
                             rows_v.at[slot, j], sems[slot])

    def wait(slot, r):
        for j in range(2):
            pltpu.make_async_copy(table_hbm.at[idx_all.at[r, j]],
                                  rows_v.at[slot, j], sems[slot]).wait()

    def reduce_store(slot, r, i):
        def seq_step(s, acc):
            acc = list(acc)
            for j in range(2):
                for k2 in range(NCHUNK // 2):
                    x32 = rows_v[slot, j, s, pl.ds(L * k2, L)]
                    a = lax.bitcast_convert_type(
                        lax.shift_left(x32, 16), jnp.float32)
                    bb = lax.bitcast_convert_type(
                        jnp.bitwise_and(x32, jnp.int32(-65536)), jnp.float32)
                    acc[2 * k2] = acc[2 * k2] + a
                    acc[2 * k2 + 1] = acc[2 * k2 + 1] + bb
            return tuple(acc)

        acc = lax.fori_loop(
            0, SEQ_HALF, seq_step,
            tuple(jnp.zeros((L,), jnp.float32) for _ in range(NCHUNK)))

        @pl.when(i > 0)
        def _():
            pltpu.make_async_copy(acc_v.at[slot], out_hbm.at[base + r],
                                  osems[slot]).wait()

        for k in range(NCHUNK):
            acc_v[slot, pl.ds(L * k, L)] = acc[k] * (1.0 / SEQ)
        pltpu.async_copy(acc_v.at[slot], out_hbm.at[base + r], osems[slot])

    issue(0, 0)

    def pair_step(i, carry):
        r0 = 2 * i
        issue(1, r0 + 1)
        wait(0, r0)
        reduce_store(0, r0, i)

        @pl.when(r0 + 2 < ROWS_PER_W)
        def _():
            issue(0, r0 + 2)

        wait(1, r0 + 1)
        reduce_store(1, r0 + 1, i)
        return carry

    lax.fori_loop(0, ROWS_PER_W // 2, pair_step, 0)

    last = ROWS_PER_W - 2
    for slot in range(2):
        pltpu.make_async_copy(acc_v.at[slot], out_hbm.at[base + last + slot],
                              osems[slot]).wait()


_pool = pl.kernel(
    _pool_body,
    out_type=jax.ShapeDtypeStruct((BATCH, EMBED), jnp.float32),
    mesh=plsc.VectorSubcoreMesh(core_axis_name="c", subcore_axis_name="s",
                                num_cores=NC, num_subcores=NS),
    compiler_params=pltpu.CompilerParams(use_tc_tiling_on_sc=False),
    scratch_types=[
        pltpu.VMEM((ROWS_PER_W, 2, SEQ_HALF), jnp.int32),
        pltpu.VMEM((2, 2, SEQ_HALF, EMBED // 2), jnp.int32),
        pltpu.VMEM((2, EMBED), jnp.float32),
        pltpu.SemaphoreType.DMA,
        pltpu.SemaphoreType.DMA,
        pltpu.SemaphoreType.DMA,
        pltpu.SemaphoreType.DMA,
    ],
)


def _linear_body(p_ref, w_ref, b_ref, o_ref):
    o_ref[...] = lax.dot_general(
        p_ref[...], w_ref[...], (((1,), (1,)), ((), ())),
        preferred_element_type=jnp.float32) + b_ref[...]


_PERM = np.concatenate(
    [np.concatenate([32 * k2 + 2 * np.arange(16),
                     32 * k2 + 2 * np.arange(16) + 1])
     for k in [0] for k2 in range(4)]).astype(np.int32)


def kernel(input_ids, table, W, b):
    ids2 = input_ids.astype(jnp.int32).reshape(BATCH, 2, SEQ_HALF)
    table_i32 = lax.bitcast_convert_type(
        table.astype(jnp.bfloat16).reshape(VOCAB, EMBED // 2, 2), jnp.int32)
    pooled = _pool(ids2, table_i32)
    W = W[:, _PERM]
    out = pl.pallas_call(
        _linear_body,
        out_shape=jax.ShapeDtypeStruct((BATCH, HIDDEN), jnp.float32),
        grid=(BATCH // 1024,),
        in_specs=[
            pl.BlockSpec((1024, EMBED), lambda i: (i, 0)),
            pl.BlockSpec((HIDDEN, EMBED), lambda i: (0, 0)),
            pl.BlockSpec((1, HIDDEN), lambda i: (0, 0)),
        ],
        out_specs=pl.BlockSpec((1024, HIDDEN), lambda i: (i, 0)),
    )(pooled, W, b.reshape(1, HIDDEN))
    return out


# speedup vs baseline: 3.1528x; 3.1528x over previous
import jax
import jax.numpy as jnp
import numpy as np
from jax import lax
from jax.experimental import pallas as pl
from jax.experimental.pallas import tpu as pltpu
from jax.experimental.pallas import tpu_sc as plsc

NC, NS, L = 2, 16, 16
NW = NC * NS

VOCAB = 100000
BATCH, SEQ = 4096, 200
EMBED, HIDDEN = 128, 128
SEQ_HALF = SEQ // 2
ROWS_PER_W = BATCH // NW
NCHUNK = EMBED // L


def _pool_body(ids_hbm, table_hbm, out_hbm, idx_all, rows_v, acc_v,
               s00, s01, s10, s11, osem0, osem1):
    wid = lax.axis_index("s") * NC + lax.axis_index("c")
    base = wid * ROWS_PER_W
    sems = ((s00, s01), (s10, s11))
    osems = (osem0, osem1)

    pltpu.sync_copy(ids_hbm.at[pl.ds(base, ROWS_PER_W)], idx_all)

    def issue(slot, r):
        for j in range(2):
            pltpu.async_copy(table_hbm.at[idx_all.at[r, j]],
                             rows_v.at[slot, j], sems[slot][j])

    def wait(slot, r, j):
        pltpu.make_async_copy(table_hbm.at[idx_all.at[r, j]],
                              rows_v.at[slot, j], sems[slot][j]).wait()

    def chunk_reduce(slot, j, acc0):
        def seq_step(s, acc):
            acc = tuple(acc[k] + rows_v[slot, j, 2 * s, pl.ds(L * k, L)]
                        for k in range(NCHUNK))
            return tuple(acc[k] + rows_v[slot, j, 2 * s + 1, pl.ds(L * k, L)]
                         for k in range(NCHUNK))

        return lax.fori_loop(0, SEQ_HALF // 2, seq_step, acc0)

    def reduce_store(slot, r, i):
        wait(slot, r, 0)
        acc = chunk_reduce(
            slot, 0, tuple(jnp.zeros((L,), jnp.float32)
                           for _ in range(NCHUNK)))
        wait(slot, r, 1)
        acc = chunk_reduce(slot, 1, acc)

        @pl.when(i > 0)
        def _():
            pltpu.make_async_copy(acc_v.at[slot], out_hbm.at[base + r],
                                  osems[slot]).wait()

        for k in range(NCHUNK):
            acc_v[slot, pl.ds(L * k, L)] = acc[k] * (1.0 / SEQ)
        pltpu.async_copy(acc_v.at[slot], out_hbm.at[base + r], osems[slot])

    issue(0, 0)

    def pair_step(i, carry):
        r0 = 2 * i
        issue(1, r0 + 1)
        reduce_store(0, r0, i)

        @pl.when(r0 + 2 < ROWS_PER_W)
        def _():
            issue(0, r0 + 2)

        reduce_store(1, r0 + 1, i)
        return carry

    lax.fori_loop(0, ROWS_PER_W // 2, pair_step, 0)

    last = ROWS_PER_W - 2
    for slot in range(2):
        pltpu.make_async_copy(acc_v.at[slot], out_hbm.at[base + last + slot],
                              osems[slot]).wait()


_pool = pl.kernel(
    _pool_body,
    out_type=jax.ShapeDtypeStruct((BATCH, EMBED), jnp.float32),
    mesh=plsc.VectorSubcoreMesh(core_axis_name="c", subcore_axis_name="s",
                                num_cores=NC, num_subcores=NS),
    scratch_types=[
        pltpu.VMEM((ROWS_PER_W, 2, SEQ_HALF), jnp.int32),
        pltpu.VMEM((2, 2, SEQ_HALF, EMBED), jnp.float32),
        pltpu.VMEM((2, EMBED), jnp.float32),
        pltpu.SemaphoreType.DMA,
        pltpu.SemaphoreType.DMA,
        pltpu.SemaphoreType.DMA,
        pltpu.SemaphoreType.DMA,
        pltpu.SemaphoreType.DMA,
        pltpu.SemaphoreType.DMA,
    ],
)


def _linear_body(p_ref, w_ref, b_ref, o_ref):
    o_ref[...] = lax.dot_general(
        p_ref[...], w_ref[...], (((1,), (1,)), ((), ())),
        preferred_element_type=jnp.float32) + b_ref[...]


def kernel(input_ids, table, W, b):
    ids2 = input_ids.astype(jnp.int32).reshape(BATCH, 2, SEQ_HALF)
    pooled = _pool(ids2, table)
    out = pl.pallas_call(
        _linear_body,
        out_shape=jax.ShapeDtypeStruct((BATCH, HIDDEN), jnp.float32),
        grid=(BATCH // 1024,),
        in_specs=[
            pl.BlockSpec((1024, EMBED), lambda i: (i, 0)),
            pl.BlockSpec((HIDDEN, EMBED), lambda i: (0, 0)),
            pl.BlockSpec((1, HIDDEN), lambda i: (0, 0)),
        ],
        out_specs=pl.BlockSpec((1024, HIDDEN), lambda i: (i, 0)),
    )(pooled, W, b.reshape(1, HIDDEN))
    return out
